# baseline (speedup 1.0000x reference)
import functools

import jax
import jax.numpy as jnp
from jax import lax
from jax.experimental import pallas as pl
from jax.experimental.pallas import tpu as pltpu
from jax.experimental.pallas import tpu_sc as plsc

VOCAB = 1000000
DIM = 64
LANE = 128
NTC = VOCAB // LANE
TAIL = VOCAB - NTC * LANE
VPAD = (NTC + 1) * LANE

NC = 2
NS = 16
NW = NC * NS

B = 4096
S = 200
NJ = B // LANE


def _relayout_body(tabT_hbm, tail_hbm, t128_hbm, tiles_v, rows_v, *sems):
    isem = sems[:2]
    osem = sems[2:]
    wid = lax.axis_index("s") * NC + lax.axis_index("c")
    iota16 = lax.iota(jnp.int32, 16)

    def transpose(b):
        def tgroup(t0, _):
            tvec = t0 * 16 + iota16
            for c in range(DIM):
                v = tiles_v[b, c, pl.ds(t0 * 16, 16)]
                plsc.store_scatter(
                    rows_v.at[b], [tvec, jnp.full((16,), c, jnp.int32)], v)
            return ()
        lax.fori_loop(0, LANE // 16, tgroup, ())

    def fire_in(b, tc):
        pltpu.async_copy(
            tabT_hbm.at[:, pl.ds(tc * LANE, LANE)], tiles_v.at[b], isem[b])

    def wait_in(b):
        pltpu.make_async_copy(
            tabT_hbm.at[:, pl.ds(0, LANE)], tiles_v.at[b], isem[b]).wait()

    def fire_out(b, tc):
        pltpu.async_copy(
            rows_v.at[b], t128_hbm.at[pl.ds(tc * LANE, LANE), :], osem[b])

    def wait_out(b):
        pltpu.make_async_copy(
            rows_v.at[b], t128_hbm.at[pl.ds(0, LANE), :], osem[b]).wait()

    n_iter = (NTC + NW - 1) // NW
    fire_in(0, wid)

    def step(i2, _):
        for b in range(2):
            tc = (2 * i2 + b) * NW + wid
            nxt = tc + NW

            @pl.when(tc < NTC)
            def _():
                wait_in(b)

                @pl.when(nxt < NTC)
                def _():
                    fire_in(1 - b, nxt)

                @pl.when(tc >= 2 * NW)
                def _():
                    wait_out(b)
                transpose(b)
                fire_out(b, tc)
        return ()

    lax.fori_loop(0, (n_iter + 1) // 2, step, ())

    wait_out(0)
    wait_out(1)

    @pl.when(wid == NW - 1)
    def _():
        pltpu.sync_copy(tail_hbm, tiles_v.at[0])
        pltpu.sync_copy(tiles_v.at[0],
                        t128_hbm.at[pl.ds(NTC * LANE, TAIL), :])


def _gather_body(idx_hbm, t128_hbm, out_hbm, idx_v, buf_v, bufT_v, *sems):
    gsem = sems[:2]
    wsem = sems[2:]
    wid = lax.axis_index("s") * NC + lax.axis_index("c")
    iota16 = lax.iota(jnp.int32, 16)

    pltpu.sync_copy(idx_hbm.at[:, wid], idx_v)

    def fire_g(b, s):
        pltpu.async_copy(
            t128_hbm.at[idx_v.at[s // 8, s % 8]], buf_v.at[b], gsem[b])

    def wait_g(b):
        pltpu.make_async_copy(
            t128_hbm.at[pl.ds(0, LANE)], buf_v.at[b], gsem[b]).wait()

    def transpose(b):
        def cgroup(g, _):
            c0 = g * 16
            chi = (c0 + iota16) // 8
            clo = (c0 + iota16) % 8
            for t in range(LANE):
                v = buf_v[b, t, pl.ds(c0, 16)]
                plsc.store_scatter(
                    bufT_v.at[b], [chi, clo, jnp.full((16,), t, jnp.int32)],
                    v)
            return ()
        lax.fori_loop(0, DIM // 16, cgroup, ())

    def fire_w(b, s):
        pltpu.async_copy(
            bufT_v.at[b], out_hbm.at[s, :, wid], wsem[b])

    def wait_w(b):
        pltpu.make_async_copy(
            bufT_v.at[b], out_hbm.at[0, :, 0], wsem[b]).wait()

    fire_g(0, 0)

    def step(i2, _):
        for b in range(2):
            s = 2 * i2 + b
            wait_g(b)

            @pl.when(s + 1 < S)
            def _():
                fire_g(1 - b, s + 1)

            @pl.when(s >= 2)
            def _():
                wait_w(b)
            transpose(b)
            fire_w(b, s)
        return ()

    lax.fori_loop(0, S // 2, step, ())
    wait_w(0)
    wait_w(1)


@jax.jit
def _embed(input_ids, table):
    mesh = plsc.VectorSubcoreMesh(core_axis_name="c", subcore_axis_name="s")
    params = pltpu.CompilerParams(
        use_tc_tiling_on_sc=True, needs_layout_passes=False)

    tabT = table.T
    tailp = jnp.pad(table[NTC * LANE:, :], ((0, 0), (0, LANE - DIM)))

    t128 = pl.kernel(
        _relayout_body,
        out_type=jax.ShapeDtypeStruct((VPAD, LANE), jnp.float32),
        mesh=mesh,
        scratch_types=(
            [pltpu.VMEM((2, DIM, LANE), jnp.float32),
             pltpu.VMEM((2, LANE, LANE), jnp.float32)]
            + [pltpu.SemaphoreType.DMA] * 4
        ),
        compiler_params=params,
    )(tabT, tailp)

    idx4 = (input_ids.astype(jnp.int32).T
            .reshape(S // 8, 8, NJ, LANE).transpose(0, 2, 1, 3))

    out5 = pl.kernel(
        _gather_body,
        out_type=jax.ShapeDtypeStruct((S, 8, NJ, 8, LANE), jnp.float32),
        mesh=mesh,
        scratch_types=(
            [pltpu.VMEM((S // 8, 8, LANE), jnp.int32),
             pltpu.VMEM((2, LANE, LANE), jnp.float32),
             pltpu.VMEM((2, 8, 8, LANE), jnp.float32)]
            + [pltpu.SemaphoreType.DMA] * 4
        ),
        compiler_params=params,
    )(idx4, t128)

    return out5.transpose(2, 4, 0, 1, 3).reshape(B, S, DIM)


def kernel(input_ids, table):
    return _embed(input_ids, table)


# speedup vs baseline: 1.3062x; 1.3062x over previous
import functools

import jax
import jax.numpy as jnp
from jax import lax
from jax.experimental import pallas as pl
from jax.experimental.pallas import tpu as pltpu
from jax.experimental.pallas import tpu_sc as plsc

VOCAB = 1000000
DIM = 64
LANE = 128
NTC = VOCAB // LANE
TAIL = VOCAB - NTC * LANE
VPAD = (NTC + 1) * LANE

NC = 2
NS = 16
NW = NC * NS

B = 4096
S = 200
NJ = B // LANE


def _relayout_body(tabT_hbm, tail_hbm, t128_hbm, tiles_v, rows_v, *sems):
    isem = sems[:2]
    osem = sems[2:]
    wid = lax.axis_index("s") * NC + lax.axis_index("c")
    iota16 = lax.iota(jnp.int32, 16)

    def transpose(b):
        def tgroup(t0, _):
            tvec = t0 * 16 + iota16
            for c0 in range(0, DIM, 16):
                vs = [tiles_v[b, c0 + j, pl.ds(t0 * 16, 16)]
                      for j in range(16)]
                for j in range(16):
                    plsc.store_scatter(
                        rows_v.at[b],
                        [tvec, jnp.full((16,), c0 + j, jnp.int32)], vs[j])
            return ()
        lax.fori_loop(0, LANE // 16, tgroup, ())

    def fire_in(b, tc):
        pltpu.async_copy(
            tabT_hbm.at[:, pl.ds(tc * LANE, LANE)], tiles_v.at[b], isem[b])

    def wait_in(b):
        pltpu.make_async_copy(
            tabT_hbm.at[:, pl.ds(0, LANE)], tiles_v.at[b], isem[b]).wait()

    def fire_out(b, tc):
        pltpu.async_copy(
            rows_v.at[b], t128_hbm.at[pl.ds(tc * LANE, LANE), :], osem[b])

    def wait_out(b):
        pltpu.make_async_copy(
            rows_v.at[b], t128_hbm.at[pl.ds(0, LANE), :], osem[b]).wait()

    n_iter = (NTC + NW - 1) // NW
    fire_in(0, wid)

    def step(i2, _):
        for b in range(2):
            tc = (2 * i2 + b) * NW + wid
            nxt = tc + NW

            @pl.when(tc < NTC)
            def _():
                wait_in(b)

                @pl.when(nxt < NTC)
                def _():
                    fire_in(1 - b, nxt)

                @pl.when(tc >= 2 * NW)
                def _():
                    wait_out(b)
                transpose(b)
                fire_out(b, tc)
        return ()

    lax.fori_loop(0, (n_iter + 1) // 2, step, ())

    wait_out(0)
    wait_out(1)

    @pl.when(wid == NW - 1)
    def _():
        pltpu.sync_copy(tail_hbm, tiles_v.at[0])
        pltpu.sync_copy(tiles_v.at[0],
                        t128_hbm.at[pl.ds(NTC * LANE, TAIL), :])


def _gather_body(idx_hbm, t128_hbm, out_hbm, idx_v, buf_v, bufT_v, *sems):
    gsem = sems[:2]
    wsem = sems[2:]
    wid = lax.axis_index("s") * NC + lax.axis_index("c")
    iota16 = lax.iota(jnp.int32, 16)

    pltpu.sync_copy(idx_hbm.at[:, wid], idx_v)

    def fire_g(b, s):
        pltpu.async_copy(
            t128_hbm.at[idx_v.at[s // 8, s % 8]], buf_v.at[b], gsem[b])

    def wait_g(b):
        pltpu.make_async_copy(
            t128_hbm.at[pl.ds(0, LANE)], buf_v.at[b], gsem[b]).wait()

    def transpose(b):
        def cgroup(g, _):
            c0 = g * 16
            chi = (c0 + iota16) // 8
            clo = (c0 + iota16) % 8
            for t0 in range(0, LANE, 16):
                vs = [buf_v[b, t0 + j, pl.ds(c0, 16)] for j in range(16)]
                for j in range(16):
                    plsc.store_scatter(
                        bufT_v.at[b],
                        [chi, clo, jnp.full((16,), t0 + j, jnp.int32)],
                        vs[j])
            return ()
        lax.fori_loop(0, DIM // 16, cgroup, ())

    def fire_w(b, s):
        pltpu.async_copy(
            bufT_v.at[b], out_hbm.at[s, :, wid], wsem[b])

    def wait_w(b):
        pltpu.make_async_copy(
            bufT_v.at[b], out_hbm.at[0, :, 0], wsem[b]).wait()

    fire_g(0, 0)

    def step(i2, _):
        for b in range(2):
            s = 2 * i2 + b
            wait_g(b)

            @pl.when(s + 1 < S)
            def _():
                fire_g(1 - b, s + 1)

            @pl.when(s >= 2)
            def _():
                wait_w(b)
            transpose(b)
            fire_w(b, s)
        return ()

    lax.fori_loop(0, S // 2, step, ())
    wait_w(0)
    wait_w(1)


@jax.jit
def _embed(input_ids, table):
    mesh = plsc.VectorSubcoreMesh(core_axis_name="c", subcore_axis_name="s")
    params = pltpu.CompilerParams(
        use_tc_tiling_on_sc=True, needs_layout_passes=False)

    tabT = table.T
    tailp = jnp.pad(table[NTC * LANE:, :], ((0, 0), (0, LANE - DIM)))

    t128 = pl.kernel(
        _relayout_body,
        out_type=jax.ShapeDtypeStruct((VPAD, LANE), jnp.float32),
        mesh=mesh,
        scratch_types=(
            [pltpu.VMEM((2, DIM, LANE), jnp.float32),
             pltpu.VMEM((2, LANE, LANE), jnp.float32)]
            + [pltpu.SemaphoreType.DMA] * 4
        ),
        compiler_params=params,
    )(tabT, tailp)

    idx4 = (input_ids.astype(jnp.int32).T
            .reshape(S // 8, 8, NJ, LANE).transpose(0, 2, 1, 3))

    out5 = pl.kernel(
        _gather_body,
        out_type=jax.ShapeDtypeStruct((S, 8, NJ, 8, LANE), jnp.float32),
        mesh=mesh,
        scratch_types=(
            [pltpu.VMEM((S // 8, 8, LANE), jnp.int32),
             pltpu.VMEM((2, LANE, LANE), jnp.float32),
             pltpu.VMEM((2, 8, 8, LANE), jnp.float32)]
            + [pltpu.SemaphoreType.DMA] * 4
        ),
        compiler_params=params,
    )(idx4, t128)

    return out5.transpose(2, 4, 0, 1, 3).reshape(B, S, DIM)


def kernel(input_ids, table):
    return _embed(input_ids, table)


# speedup vs baseline: 3.3218x; 2.5432x over previous
import functools

import jax
import jax.numpy as jnp
from jax import lax
from jax.experimental import pallas as pl
from jax.experimental.pallas import tpu as pltpu
from jax.experimental.pallas import tpu_sc as plsc

VOCAB = 1000000
DIM = 64
LANE = 128
NTC = VOCAB // LANE
TAIL = VOCAB - NTC * LANE
VPAD = (NTC + 1) * LANE

NC = 2
NS = 16
NW = NC * NS

B = 4096
S = 200
NJ = B // LANE


def _relayout_body(tabT_hbm, tail_hbm, t128_hbm, tiles_v, rows_v, *sems):
    isem = sems[:2]
    osem = sems[2:]
    wid = lax.axis_index("s") * NC + lax.axis_index("c")
    iota16 = lax.iota(jnp.int32, 16)

    def transpose(b):
        def tgroup(t0, _):
            tvec = t0 * 16 + iota16
            for c0 in range(0, DIM, 16):
                cvs = [c0 + ((iota16 + d) & 15) for d in range(16)]
                vs = [plsc.load_gather(tiles_v.at[b], [cvs[d], tvec])
                      for d in range(16)]
                for d in range(16):
                    plsc.store_scatter(rows_v.at[b], [tvec, cvs[d]], vs[d])
            return ()
        lax.fori_loop(0, LANE // 16, tgroup, ())

    def fire_in(b, tc):
        pltpu.async_copy(
            tabT_hbm.at[:, pl.ds(tc * LANE, LANE)], tiles_v.at[b], isem[b])

    def wait_in(b):
        pltpu.make_async_copy(
            tabT_hbm.at[:, pl.ds(0, LANE)], tiles_v.at[b], isem[b]).wait()

    def fire_out(b, tc):
        pltpu.async_copy(
            rows_v.at[b], t128_hbm.at[pl.ds(tc * LANE, LANE), :], osem[b])

    def wait_out(b):
        pltpu.make_async_copy(
            rows_v.at[b], t128_hbm.at[pl.ds(0, LANE), :], osem[b]).wait()

    n_iter = (NTC + NW - 1) // NW
    fire_in(0, wid)

    def step(i2, _):
        for b in range(2):
            tc = (2 * i2 + b) * NW + wid
            nxt = tc + NW

            @pl.when(tc < NTC)
            def _():
                wait_in(b)

                @pl.when(nxt < NTC)
                def _():
                    fire_in(1 - b, nxt)

                @pl.when(tc >= 2 * NW)
                def _():
                    wait_out(b)
                transpose(b)
                fire_out(b, tc)
        return ()

    lax.fori_loop(0, (n_iter + 1) // 2, step, ())

    wait_out(0)
    wait_out(1)

    @pl.when(wid == NW - 1)
    def _():
        pltpu.sync_copy(tail_hbm, tiles_v.at[0])
        pltpu.sync_copy(tiles_v.at[0],
                        t128_hbm.at[pl.ds(NTC * LANE, TAIL), :])


def _gather_body(idx_hbm, t128_hbm, out_hbm, idx_v, buf_v, bufT_v, *sems):
    gsem = sems[:2]
    wsem = sems[2:]
    wid = lax.axis_index("s") * NC + lax.axis_index("c")
    iota16 = lax.iota(jnp.int32, 16)

    pltpu.sync_copy(idx_hbm.at[:, wid], idx_v)

    def fire_g(b, s):
        pltpu.async_copy(
            t128_hbm.at[idx_v.at[s // 8, s % 8]], buf_v.at[b], gsem[b])

    def wait_g(b):
        pltpu.make_async_copy(
            t128_hbm.at[pl.ds(0, LANE)], buf_v.at[b], gsem[b]).wait()

    def transpose(b):
        def tgroup(t0, _):
            tvec = t0 * 16 + iota16
            for c0 in range(0, DIM, 16):
                cvs = [c0 + ((iota16 + d) & 15) for d in range(16)]
                vs = [plsc.load_gather(buf_v.at[b], [tvec, cvs[d]])
                      for d in range(16)]
                for d in range(16):
                    plsc.store_scatter(
                        bufT_v.at[b], [cvs[d] >> 3, cvs[d] & 7, tvec],
                        vs[d])
            return ()
        lax.fori_loop(0, LANE // 16, tgroup, ())

    def fire_w(b, s):
        pltpu.async_copy(
            bufT_v.at[b], out_hbm.at[s, :, wid], wsem[b])

    def wait_w(b):
        pltpu.make_async_copy(
            bufT_v.at[b], out_hbm.at[0, :, 0], wsem[b]).wait()

    fire_g(0, 0)

    def step(i2, _):
        for b in range(2):
            s = 2 * i2 + b
            wait_g(b)

            @pl.when(s + 1 < S)
            def _():
                fire_g(1 - b, s + 1)

            @pl.when(s >= 2)
            def _():
                wait_w(b)
            transpose(b)
            fire_w(b, s)
        return ()

    lax.fori_loop(0, S // 2, step, ())
    wait_w(0)
    wait_w(1)


@jax.jit
def _embed(input_ids, table):
    mesh = plsc.VectorSubcoreMesh(core_axis_name="c", subcore_axis_name="s")
    params = pltpu.CompilerParams(
        use_tc_tiling_on_sc=True, needs_layout_passes=False)

    tabT = table.T
    tailp = jnp.pad(table[NTC * LANE:, :], ((0, 0), (0, LANE - DIM)))

    t128 = pl.kernel(
        _relayout_body,
        out_type=jax.ShapeDtypeStruct((VPAD, LANE), jnp.float32),
        mesh=mesh,
        scratch_types=(
            [pltpu.VMEM((2, DIM, LANE), jnp.float32),
             pltpu.VMEM((2, LANE, LANE), jnp.float32)]
            + [pltpu.SemaphoreType.DMA] * 4
        ),
        compiler_params=params,
    )(tabT, tailp)

    idx4 = (input_ids.astype(jnp.int32).T
            .reshape(S // 8, 8, NJ, LANE).transpose(0, 2, 1, 3))

    out5 = pl.kernel(
        _gather_body,
        out_type=jax.ShapeDtypeStruct((S, 8, NJ, 8, LANE), jnp.float32),
        mesh=mesh,
        scratch_types=(
            [pltpu.VMEM((S // 8, 8, LANE), jnp.int32),
             pltpu.VMEM((2, LANE, LANE), jnp.float32),
             pltpu.VMEM((2, 8, 8, LANE), jnp.float32)]
            + [pltpu.SemaphoreType.DMA] * 4
        ),
        compiler_params=params,
    )(idx4, t128)

    return out5.transpose(2, 4, 0, 1, 3).reshape(B, S, DIM)


def kernel(input_ids, table):
    return _embed(input_ids, table)


# speedup vs baseline: 3.4947x; 1.0520x over previous
import functools

import jax
import jax.numpy as jnp
from jax import lax
from jax.experimental import pallas as pl
from jax.experimental.pallas import tpu as pltpu
from jax.experimental.pallas import tpu_sc as plsc

VOCAB = 1000000
DIM = 64
LANE = 128
NTC = VOCAB // LANE
VP = VOCAB // 2

NC = 2
NS = 16
NW = NC * NS

B = 4096
S = 200
NJ = B // LANE


def _relayout_body(tabT_hbm, tail_hbm, t2_hbm, tiles_v, rows_v, *sems):
    isem = sems[:2]
    osem = sems[2:]
    wid = lax.axis_index("s") * NC + lax.axis_index("c")
    iota16 = lax.iota(jnp.int32, 16)

    def transpose(b):
        def tgroup(t0, _):
            tvec = t0 * 16 + iota16
            qvec = tvec >> 1
            hvec = (tvec & 1) << 6
            for c0 in range(0, DIM, 16):
                cvs = [c0 + ((iota16 + d) & 15) for d in range(16)]
                vs = [plsc.load_gather(tiles_v.at[b], [cvs[d], tvec])
                      for d in range(16)]
                for d in range(16):
                    plsc.store_scatter(
                        rows_v.at[b], [qvec, hvec + cvs[d]], vs[d])
            return ()
        lax.fori_loop(0, LANE // 16, tgroup, ())

    def fire_in(b, tc):
        pltpu.async_copy(
            tabT_hbm.at[:, pl.ds(tc * LANE, LANE)], tiles_v.at[b], isem[b])

    def wait_in(b):
        pltpu.make_async_copy(
            tabT_hbm.at[:, pl.ds(0, LANE)], tiles_v.at[b], isem[b]).wait()

    def fire_out(b, tc):
        pltpu.async_copy(
            rows_v.at[b], t2_hbm.at[pl.ds(tc * DIM, DIM), :], osem[b])

    def wait_out(b):
        pltpu.make_async_copy(
            rows_v.at[b], t2_hbm.at[pl.ds(0, DIM), :], osem[b]).wait()

    n_iter = (NTC + NW - 1) // NW
    fire_in(0, wid)

    def step(i2, _):
        for b in range(2):
            tc = (2 * i2 + b) * NW + wid
            nxt = tc + NW

            @pl.when(tc < NTC)
            def _():
                wait_in(b)

                @pl.when(nxt < NTC)
                def _():
                    fire_in(1 - b, nxt)

                @pl.when(tc >= 2 * NW)
                def _():
                    wait_out(b)
                transpose(b)
                fire_out(b, tc)
        return ()

    lax.fori_loop(0, (n_iter + 1) // 2, step, ())

    wait_out(0)
    wait_out(1)

    @pl.when(wid == NW - 1)
    def _():
        pltpu.sync_copy(tail_hbm, tiles_v.at[0, pl.ds(0, 32), :])
        pltpu.sync_copy(tiles_v.at[0, pl.ds(0, 32), :],
                        t2_hbm.at[pl.ds(NTC * DIM, 32), :])


def _gather_body(idx_hbm, t2_hbm, out_hbm, idx_v, idxq_v, buf_v, bufT_v,
                 *sems):
    gsem = sems[:2]
    wsem = sems[2:]
    wid = lax.axis_index("s") * NC + lax.axis_index("c")
    iota16 = lax.iota(jnp.int32, 16)

    pltpu.sync_copy(idx_hbm.at[:, wid], idx_v)

    def halve(a, _):
        for k in range(8):
            for c0 in range(0, LANE, 16):
                v = idx_v[a, k, pl.ds(c0, 16)]
                idxq_v[a, k, pl.ds(c0, 16)] = v >> 1
        return ()
    lax.fori_loop(0, S // 8, halve, ())

    def fire_g(b, s):
        pltpu.async_copy(
            t2_hbm.at[idxq_v.at[s // 8, s % 8]], buf_v.at[b], gsem[b])

    def wait_g(b):
        pltpu.make_async_copy(
            t2_hbm.at[pl.ds(0, LANE)], buf_v.at[b], gsem[b]).wait()

    def transpose(b, s):
        a = s // 8
        k = s % 8

        def tgroup(t0, _):
            tvec = t0 * 16 + iota16
            offs = (idx_v[a, k, pl.ds(t0 * 16, 16)] & 1) << 6
            for c0 in range(0, DIM, 16):
                cvs = [c0 + ((iota16 + d) & 15) for d in range(16)]
                vs = [plsc.load_gather(buf_v.at[b], [tvec, cvs[d] + offs])
                      for d in range(16)]
                for d in range(16):
                    plsc.store_scatter(
                        bufT_v.at[b], [cvs[d] >> 3, cvs[d] & 7, tvec],
                        vs[d])
            return ()
        lax.fori_loop(0, LANE // 16, tgroup, ())

    def fire_w(b, s):
        pltpu.async_copy(
            bufT_v.at[b], out_hbm.at[s, :, wid], wsem[b])

    def wait_w(b):
        pltpu.make_async_copy(
            bufT_v.at[b], out_hbm.at[0, :, 0], wsem[b]).wait()

    fire_g(0, 0)

    def step(i2, _):
        for b in range(2):
            s = 2 * i2 + b
            wait_g(b)

            @pl.when(s + 1 < S)
            def _():
                fire_g(1 - b, s + 1)

            @pl.when(s >= 2)
            def _():
                wait_w(b)
            transpose(b, s)
            fire_w(b, s)
        return ()

    lax.fori_loop(0, S // 2, step, ())
    wait_w(0)
    wait_w(1)


@jax.jit
def _embed(input_ids, table):
    mesh = plsc.VectorSubcoreMesh(core_axis_name="c", subcore_axis_name="s")
    params = pltpu.CompilerParams(
        use_tc_tiling_on_sc=True, needs_layout_passes=False)

    tabT = table.T
    tail2 = table[NTC * LANE:, :].reshape(32, LANE)

    t2 = pl.kernel(
        _relayout_body,
        out_type=jax.ShapeDtypeStruct((VP, LANE), jnp.float32),
        mesh=mesh,
        scratch_types=(
            [pltpu.VMEM((2, DIM, LANE), jnp.float32),
             pltpu.VMEM((2, DIM, LANE), jnp.float32)]
            + [pltpu.SemaphoreType.DMA] * 4
        ),
        compiler_params=params,
    )(tabT, tail2)

    idx4 = (input_ids.astype(jnp.int32).T
            .reshape(S // 8, 8, NJ, LANE).transpose(0, 2, 1, 3))

    out5 = pl.kernel(
        _gather_body,
        out_type=jax.ShapeDtypeStruct((S, 8, NJ, 8, LANE), jnp.float32),
        mesh=mesh,
        scratch_types=(
            [pltpu.VMEM((S // 8, 8, LANE), jnp.int32),
             pltpu.VMEM((S // 8, 8, LANE), jnp.int32),
             pltpu.VMEM((2, LANE, LANE), jnp.float32),
             pltpu.VMEM((2, 8, 8, LANE), jnp.float32)]
            + [pltpu.SemaphoreType.DMA] * 4
        ),
        compiler_params=params,
    )(idx4, t2)

    return out5.transpose(2, 4, 0, 1, 3).reshape(B, S, DIM)


def kernel(input_ids, table):
    return _embed(input_ids, table)


# speedup vs baseline: 3.5154x; 1.0059x over previous
import jax
import jax.numpy as jnp
from jax import lax
from jax.experimental import pallas as pl
from jax.experimental.pallas import tpu as pltpu
from jax.experimental.pallas import tpu_sc as plsc

VOCAB = 1000000
DIM = 64
LANE = 128
NTC = VOCAB // LANE
VP = VOCAB // 2

NC = 2
NS = 16
NW = NC * NS

B = 4096
S = 200
NJ = B // LANE


def _relayout_body(tabT_hbm, tail_hbm, t2_hbm, tiles_v, rows_v, *sems):
    isem = sems[:2]
    osem = sems[2:]
    wid = lax.axis_index("s") * NC + lax.axis_index("c")
    iota16 = lax.iota(jnp.int32, 16)

    def transpose(b):
        def tgroup(t0, _):
            tvec = t0 * 16 + iota16
            qvec = tvec >> 1
            hvec = (tvec & 1) << 6
            for c0 in range(0, DIM, 16):
                cvs = [c0 + ((iota16 + d) & 15) for d in range(16)]
                vs = [plsc.load_gather(tiles_v.at[b], [cvs[d], tvec])
                      for d in range(16)]
                for d in range(16):
                    plsc.store_scatter(
                        rows_v.at[b], [qvec, hvec + cvs[d]], vs[d])
            return ()
        lax.fori_loop(0, LANE // 16, tgroup, ())

    def fire_in(b, tc):
        pltpu.async_copy(
            tabT_hbm.at[:, pl.ds(tc * LANE, LANE)], tiles_v.at[b], isem[b])

    def wait_in(b):
        pltpu.make_async_copy(
            tabT_hbm.at[:, pl.ds(0, LANE)], tiles_v.at[b], isem[b]).wait()

    def fire_out(b, tc):
        pltpu.async_copy(
            rows_v.at[b], t2_hbm.at[pl.ds(tc * DIM, DIM), :], osem[b])

    def wait_out(b):
        pltpu.make_async_copy(
            rows_v.at[b], t2_hbm.at[pl.ds(0, DIM), :], osem[b]).wait()

    n_iter = (NTC + NW - 1) // NW
    fire_in(0, wid)

    def step(i2, _):
        for b in range(2):
            tc = (2 * i2 + b) * NW + wid
            nxt = tc + NW

            @pl.when(tc < NTC)
            def _():
                wait_in(b)

                @pl.when(nxt < NTC)
                def _():
                    fire_in(1 - b, nxt)

                @pl.when(tc >= 2 * NW)
                def _():
                    wait_out(b)
                transpose(b)
                fire_out(b, tc)
        return ()

    lax.fori_loop(0, (n_iter + 1) // 2, step, ())

    wait_out(0)
    wait_out(1)

    @pl.when(wid == NW - 1)
    def _():
        pltpu.sync_copy(tail_hbm, tiles_v.at[0, pl.ds(0, 32), :])
        pltpu.sync_copy(tiles_v.at[0, pl.ds(0, 32), :],
                        t2_hbm.at[pl.ds(NTC * DIM, 32), :])


def _gather_body(idx_hbm, t2_hbm, out_hbm, idx_v, idxq_v, buf_v, bufT_v,
                 *sems):
    gsem = sems[:2]
    wsem = sems[2:]
    wid = lax.axis_index("s") * NC + lax.axis_index("c")
    iota16 = lax.iota(jnp.int32, 16)

    pltpu.sync_copy(idx_hbm.at[:, wid], idx_v)

    def halve(a, _):
        for k in range(8):
            for c0 in range(0, LANE, 16):
                v = idx_v[a, k, pl.ds(c0, 16)]
                idxq_v[a, k, pl.ds(c0, 16)] = v >> 1
        return ()
    lax.fori_loop(0, S // 8, halve, ())

    def fire_g(b, s):
        pltpu.async_copy(
            t2_hbm.at[idxq_v.at[s // 8, s % 8]], buf_v.at[b], gsem[b])

    def wait_g(b):
        pltpu.make_async_copy(
            t2_hbm.at[pl.ds(0, LANE)], buf_v.at[b], gsem[b]).wait()

    def transpose(b, s):
        a = s // 8
        k = s % 8

        def tgroup(t0, _):
            tvec = t0 * 16 + iota16
            offs = (idx_v[a, k, pl.ds(t0 * 16, 16)] & 1) << 6
            for c0 in range(0, DIM, 16):
                cvs = [c0 + ((iota16 + d) & 15) for d in range(16)]
                vs = [plsc.load_gather(buf_v.at[b], [tvec, cvs[d] + offs])
                      for d in range(16)]
                for d in range(16):
                    plsc.store_scatter(
                        bufT_v.at[b], [cvs[d] >> 3, cvs[d] & 7, tvec],
                        vs[d])
            return ()
        lax.fori_loop(0, LANE // 16, tgroup, ())

    def fire_w(b, s):
        pltpu.async_copy(
            bufT_v.at[b], out_hbm.at[s, :, wid], wsem[b])

    def wait_w(b):
        pltpu.make_async_copy(
            bufT_v.at[b], out_hbm.at[0, :, 0], wsem[b]).wait()

    fire_g(0, 0)

    def step(i2, _):
        for b in range(2):
            s = 2 * i2 + b
            wait_g(b)

            @pl.when(s + 1 < S)
            def _():
                fire_g(1 - b, s + 1)

            @pl.when(s >= 2)
            def _():
                wait_w(b)
            transpose(b, s)
            fire_w(b, s)
        return ()

    lax.fori_loop(0, S // 2, step, ())
    wait_w(0)
    wait_w(1)


@jax.jit
def _embed(input_ids, table):
    mesh = plsc.VectorSubcoreMesh(core_axis_name="c", subcore_axis_name="s")
    params = pltpu.CompilerParams(
        use_tc_tiling_on_sc=True, needs_layout_passes=False)

    tabT = table.T
    tail2 = table[NTC * LANE:, :].reshape(32, LANE)

    t2 = pl.kernel(
        _relayout_body,
        out_type=jax.ShapeDtypeStruct((VP, LANE), jnp.float32),
        mesh=mesh,
        scratch_types=(
            [pltpu.VMEM((2, DIM, LANE), jnp.float32),
             pltpu.VMEM((2, DIM, LANE), jnp.float32)]
            + [pltpu.SemaphoreType.DMA] * 4
        ),
        compiler_params=params,
    )(tabT, tail2)

    idx4 = (input_ids.astype(jnp.int32).T
            .reshape(S // 8, 8, NJ, LANE).transpose(0, 2, 1, 3))

    out5 = pl.kernel(
        _gather_body,
        out_type=jax.ShapeDtypeStruct((S, 8, NJ, 8, LANE), jnp.float32),
        mesh=mesh,
        scratch_types=(
            [pltpu.VMEM((S // 8, 8, LANE), jnp.int32),
             pltpu.VMEM((S // 8, 8, LANE), jnp.int32),
             pltpu.VMEM((2, LANE, LANE), jnp.float32),
             pltpu.VMEM((2, 8, 8, LANE), jnp.float32)]
            + [pltpu.SemaphoreType.DMA] * 4
        ),
        compiler_params=params,
    )(idx4, t2)

    return out5.transpose(2, 4, 0, 1, 3).reshape(B, S, DIM)


def kernel(input_ids, table):
    return _embed(input_ids, table)
